# baseline (speedup 1.0000x reference)
import functools

import jax
import jax.numpy as jnp
from jax import lax
from jax.experimental import pallas as pl
from jax.experimental.pallas import tpu as pltpu, tpu_sc as plsc

B = 16384
D = 128
VOCABS = [4, 2, 2, 5, 3, 4, 4]
NT = len(VOCABS)
NROWS = sum(VOCABS)
RPAD = 32
OFFS = [0]
for _v in VOCABS[:-1]:
    OFFS.append(OFFS[-1] + _v)

NC = 2
NS = 16
NW = NC * NS
BPW = B // NW
LANES = 16
NCHUNK = BPW // LANES
DCHUNK = D // LANES
MUNROLL = 2


def _tanh16(x):
    ax = jnp.abs(x)
    e = jnp.exp(-2.0 * ax)
    return jnp.sign(x) * ((1.0 - e) / (1.0 + e))


def _table_of(r):
    t = jnp.int32(0)
    for i in range(1, NT):
        t = t + (r >= OFFS[i]).astype(jnp.int32)
    return t


def _row_dot(erow, wrow):
    acc = None
    for k in range(DCHUNK):
        e = erow[pl.ds(k * LANES, LANES)]
        w = wrow[pl.ds(k * LANES, LANES)]
        term = _tanh16(e) * w
        acc = term if acc is None else acc + term
    return jnp.sum(acc)


def _sc_body(x_hbm, e_hbm, w_hbm, out_hbm,
             xv, er1, er2, wr1, wr2, tmp, stv, sv, outv, shared, sem):
    cid = lax.axis_index("c")
    sid = lax.axis_index("s")
    wid = sid * NC + cid
    base = wid * BPW

    r1 = sid
    r2 = jnp.minimum(jnp.int32(16) + sid, jnp.int32(NROWS - 1))
    t1 = _table_of(r1)
    t2 = _table_of(r2)

    copies = [
        pltpu.async_copy(x_hbm.at[:, pl.ds(base, BPW)], xv, sem),
        pltpu.async_copy(e_hbm.at[pl.ds(r1 * D, D)], er1, sem),
        pltpu.async_copy(e_hbm.at[pl.ds(r2 * D, D)], er2, sem),
        pltpu.async_copy(w_hbm.at[pl.ds(t1 * D, D)], wr1, sem),
        pltpu.async_copy(w_hbm.at[pl.ds(t2 * D, D)], wr2, sem),
    ]
    for c in copies:
        c.wait()

    s1 = _row_dot(er1, wr1)
    s2 = jnp.where(sid < 8, _row_dot(er2, wr2), 0.0)
    lane = lax.iota(jnp.int32, LANES)
    svec = jnp.where(lane == 0, s1, jnp.where(lane == 1, s2, 0.0))
    tmp[...] = svec
    pltpu.sync_copy(tmp, shared.at[pl.ds(sid * LANES, LANES)])
    plsc.subcore_barrier()

    pltpu.sync_copy(shared, stv)
    sv[pl.ds(0, LANES)] = plsc.load_gather(stv, [lane * LANES])
    sv[pl.ds(LANES, LANES)] = plsc.load_gather(stv, [lane * LANES + 1])

    def chunk_body(j, carry):
        for u in range(MUNROLL):
            off = (j * MUNROLL + u) * LANES
            acc = None
            for i in range(NT):
                idx = xv[i, pl.ds(off, LANES)] + OFFS[i]
                g = plsc.load_gather(sv, [idx])
                acc = g if acc is None else acc + g
            outv[pl.ds(off, LANES)] = acc
        return carry

    lax.fori_loop(0, NCHUNK // MUNROLL, chunk_body, 0)

    pltpu.sync_copy(outv, out_hbm.at[pl.ds(base, BPW)])


@jax.jit
def _run(x, epk, w):
    mesh = plsc.VectorSubcoreMesh(core_axis_name="c", subcore_axis_name="s")
    f = functools.partial(
        pl.kernel,
        mesh=mesh,
        out_type=jax.ShapeDtypeStruct((B,), jnp.float32),
        scratch_types=[
            pltpu.VMEM((NT, BPW), jnp.int32),
            pltpu.VMEM((D,), jnp.float32),
            pltpu.VMEM((D,), jnp.float32),
            pltpu.VMEM((D,), jnp.float32),
            pltpu.VMEM((D,), jnp.float32),
            pltpu.VMEM((LANES,), jnp.float32),
            pltpu.VMEM((NS * LANES,), jnp.float32),
            pltpu.VMEM((RPAD,), jnp.float32),
            pltpu.VMEM((BPW,), jnp.float32),
            pltpu.VMEM_SHARED((NS * LANES,), jnp.float32),
            pltpu.SemaphoreType.DMA,
        ],
        compiler_params=pltpu.CompilerParams(needs_layout_passes=False),
    )(_sc_body)
    return f(x, epk, w)


def kernel(input, E1, E2, E3, E4, E5, E6, E7, W):
    epk = jnp.concatenate(
        [E1, E2, E3, E4, E5, E6, E7], axis=0
    ).reshape(-1)
    out = _run(input, epk, W.reshape(-1))
    return out.reshape(B, 1)
